# baseline (speedup 1.0000x reference)
import jax
import jax.numpy as jnp
from jax.experimental import pallas as pl

PER_STEP = 16

_BF = jnp.bfloat16


def _dotT(w, x):
    return jax.lax.dot_general(w, x, (((0,), (0,)), ((), ())),
                               preferred_element_type=jnp.float32)


def _shift1(p):
    return jnp.concatenate(
        [jnp.zeros((p.shape[0], 1), p.dtype), p[:, :-1]], axis=1)


def _geoconv_kernel(c_ref, s_ref, w1f_ref, wn1_ref, w2p_ref, w2s_ref, wn2_ref,
                    wc3_ref, bcat_ref, out_ref):
    w1p = w1f_ref[:, :10]
    wpr = w1f_ref[:, 138:154]
    w1sn = jnp.concatenate(
        [w1f_ref[:, 10:138], wn1_ref[...].astype(_BF)], axis=0)
    w2p = w2p_ref[...].astype(_BF)
    w2sn = jnp.concatenate(
        [w2s_ref[...], wn2_ref[...]], axis=0).astype(_BF)
    wc3 = wc3_ref[...]
    b2p = bcat_ref[0:128]
    b2c = bcat_ref[128:138]
    bpr = bcat_ref[138:154]
    bc = bcat_ref[154:186]

    for i in range(PER_STEP):
        ct = c_ref[i]
        s = s_ref[i].astype(_BF)
        X = jnp.concatenate([ct, s, jnp.ones_like(s)], axis=0)
        L = X.shape[1]

        p1 = jax.nn.relu(_dotT(w1p, X)).astype(_BF)
        h1 = _dotT(w1sn, jnp.concatenate([X, _shift1(p1)], axis=0)
                   ).astype(_BF)

        p2 = jax.nn.relu(_dotT(w2p, h1) + b2p).astype(_BF)
        h2 = (_dotT(w2sn, jnp.concatenate([h1, _shift1(p2)], axis=0))
              + b2c).astype(_BF)

        ci = jnp.tanh(_dotT(wpr, h2) + bpr).astype(_BF)

        zc = jnp.zeros((ci.shape[0], 2), _BF)
        ci3 = jnp.concatenate(
            [ci,
             jnp.concatenate([ci[:, 1:], zc[:, :1]], axis=1),
             jnp.concatenate([ci[:, 2:], zc], axis=1)], axis=0)
        y = _dotT(wc3, ci3) + bc

        out_ref[i] = jnp.where(y > 0, y, jnp.exp(y) - 1.0)[:, : L - 2]


def _fold_emb(W, b, emb):
    de = emb[1] - emb[0]
    v = de[0] * W[8] + de[1] * W[9]
    c = emb[0, 0] * W[8] + emb[0, 1] * W[9] + b
    return jnp.concatenate([W[:8], v[None, :], c[None, :]], axis=0)


def kernel(coords, current_state, emb, Wpool1, bpool1, Wself1, Wneigh1, b1,
           Wpool2, bpool2, Wself2, Wneigh2, b2, Wproc, bproc, Wconv, bconv):
    B, L, _ = coords.shape
    C = Wconv.shape[0]
    P = PER_STEP

    ctr = jnp.transpose(coords, (0, 2, 1)).astype(_BF)
    s3 = current_state.reshape(B, 1, L)
    w1f = jnp.concatenate(
        [_fold_emb(Wpool1, bpool1, emb), _fold_emb(Wself1, b1, emb), Wproc],
        axis=1).astype(_BF)
    wc3 = jnp.transpose(Wconv, (2, 1, 0)).reshape(3 * 16, C).astype(_BF)
    bcat = jnp.concatenate([bpool2, b2, bproc, bconv])[:, None]

    full = lambda shape: pl.BlockSpec(shape, lambda b: (0,) * len(shape))
    grid_spec = pl.GridSpec(
        grid=(B // P,),
        in_specs=[
            pl.BlockSpec((P, 8, L), lambda b: (b, 0, 0)),
            pl.BlockSpec((P, 1, L), lambda b: (b, 0, 0)),
            full(w1f.shape), full(Wneigh1.shape), full(Wpool2.shape),
            full(Wself2.shape), full(Wneigh2.shape),
            full(wc3.shape), full(bcat.shape),
        ],
        out_specs=pl.BlockSpec((P, C, L - 2), lambda b: (b, 0, 0)),
    )
    out = pl.pallas_call(
        _geoconv_kernel,
        grid_spec=grid_spec,
        out_shape=jax.ShapeDtypeStruct((B, C, L - 2), jnp.float32),
    )(ctr, s3, w1f, Wneigh1, Wpool2, Wself2, Wneigh2, wc3, bcat)
    return out


# speedup vs baseline: 1.0540x; 1.0540x over previous
import jax
import jax.numpy as jnp
from jax.experimental import pallas as pl

PER_STEP = 8

_BF = jnp.bfloat16


def _dotT(w, x):
    return jax.lax.dot_general(w, x, (((0,), (0,)), ((), ())),
                               preferred_element_type=jnp.float32)


def _shift1(p):
    return jnp.concatenate(
        [jnp.zeros((p.shape[0], 1), p.dtype), p[:, :-1]], axis=1)


def _geoconv_kernel(c_ref, s_ref, w1f_ref, wn1_ref, w2p_ref, w2sn_ref,
                    wc3_ref, bcat_ref, out_ref):
    w1p = w1f_ref[:, :10]
    wpr = w1f_ref[:, 138:154]
    w1sn = jnp.concatenate(
        [w1f_ref[:, 10:138], wn1_ref[...].astype(_BF)], axis=0)
    w2p = w2p_ref[...].astype(_BF)
    w2sn = w2sn_ref[...].astype(_BF)
    wc3 = wc3_ref[...]
    b2p = bcat_ref[0:128]
    b2c = bcat_ref[128:138]
    bpr = bcat_ref[138:154]
    bc = bcat_ref[154:186]

    for i in range(PER_STEP):
        ct = c_ref[i]
        s = s_ref[i].astype(_BF)
        X = jnp.concatenate([ct, s, jnp.ones_like(s)], axis=0)
        L = X.shape[1]

        p1 = jax.nn.relu(_dotT(w1p, X)).astype(_BF)
        h1 = _dotT(w1sn, jnp.concatenate([X, _shift1(p1)], axis=0)
                   ).astype(_BF)

        p2 = jax.nn.relu(_dotT(w2p, h1) + b2p).astype(_BF)
        h2 = (_dotT(w2sn, jnp.concatenate([h1, _shift1(p2)], axis=0))
              + b2c).astype(_BF)

        ci = jnp.tanh(_dotT(wpr, h2) + bpr).astype(_BF)

        zc = jnp.zeros((ci.shape[0], 2), _BF)
        ci3 = jnp.concatenate(
            [ci,
             jnp.concatenate([ci[:, 1:], zc[:, :1]], axis=1),
             jnp.concatenate([ci[:, 2:], zc], axis=1)], axis=0)
        y = _dotT(wc3, ci3) + bc

        out_ref[i] = jnp.where(y > 0, y, jnp.exp(y) - 1.0)[:, : L - 2]


def _fold_emb(W, b, emb):
    de = emb[1] - emb[0]
    v = de[0] * W[8] + de[1] * W[9]
    c = emb[0, 0] * W[8] + emb[0, 1] * W[9] + b
    return jnp.concatenate([W[:8], v[None, :], c[None, :]], axis=0)


def kernel(coords, current_state, emb, Wpool1, bpool1, Wself1, Wneigh1, b1,
           Wpool2, bpool2, Wself2, Wneigh2, b2, Wproc, bproc, Wconv, bconv):
    B, L, _ = coords.shape
    C = Wconv.shape[0]
    P = PER_STEP

    ctr = jnp.transpose(coords, (0, 2, 1)).astype(_BF)
    s3 = current_state.reshape(B, 1, L)
    w1f = jnp.concatenate(
        [_fold_emb(Wpool1, bpool1, emb), _fold_emb(Wself1, b1, emb), Wproc],
        axis=1).astype(_BF)
    w2sn = jnp.concatenate([Wself2, Wneigh2], axis=0)
    wc3 = jnp.transpose(Wconv, (2, 1, 0)).reshape(3 * 16, C).astype(_BF)
    bcat = jnp.concatenate([bpool2, b2, bproc, bconv])[:, None]

    full = lambda shape: pl.BlockSpec(shape, lambda b: (0,) * len(shape))
    grid_spec = pl.GridSpec(
        grid=(B // P,),
        in_specs=[
            pl.BlockSpec((P, 8, L), lambda b: (b, 0, 0)),
            pl.BlockSpec((P, 1, L), lambda b: (b, 0, 0)),
            full(w1f.shape), full(Wneigh1.shape), full(Wpool2.shape),
            full(w2sn.shape), full(wc3.shape), full(bcat.shape),
        ],
        out_specs=pl.BlockSpec((P, C, L - 2), lambda b: (b, 0, 0)),
    )
    out = pl.pallas_call(
        _geoconv_kernel,
        grid_spec=grid_spec,
        out_shape=jax.ShapeDtypeStruct((B, C, L - 2), jnp.float32),
    )(ctr, s3, w1f, Wneigh1, Wpool2, w2sn, wc3, bcat)
    return out
